# Initial kernel scaffold; baseline (speedup 1.0000x reference)
import jax
import jax.numpy as jnp
from jax.experimental import pallas as pl


def kernel(x, w_ih1f, w_hh1f, b_ih1f, b_hh1f, w_ih1b, w_hh1b, b_ih1b, b_hh1b, w_ih2f, w_hh2f, b_ih2f, b_hh2f, w_ih2b, w_hh2b, b_ih2b, b_hh2b, w_lin, b_lin):
    raise NotImplementedError("write your pallas kernel here")


# speedup vs baseline: 2.6475x; 2.6475x over previous
import jax
import jax.numpy as jnp
from jax.experimental import pallas as pl
from jax.experimental.pallas import tpu as pltpu

H = 32
B = 64
M = 64
L = 256
N = M * B
G4 = 4 * H


def _sig(v):
    return jax.nn.sigmoid(v)


def _gates(g):
    return (_sig(g[0:H]), _sig(g[H:2 * H]), jnp.tanh(g[2 * H:3 * H]),
            _sig(g[3 * H:4 * H]))


def _lstm1_kernel(xT_ref, whh_ref, wih_ref, b1_ref, wihb_ref, b1b_ref, a_ref):
    bn = xT_ref.shape[1]
    whh = whh_ref[...]
    wih = wih_ref[...]
    b1 = b1_ref[...]
    h0 = jnp.zeros((H, bn), jnp.float32)

    def chunk(k, carry):
        h, c = carry
        xc = xT_ref[pl.ds(pl.multiple_of(k * 8, 8), 8), :]
        for s in range(8):
            xt = xc[s:s + 1, :]
            g = jnp.dot(whh, h, preferred_element_type=jnp.float32)
            g = g + (wih * xt + b1)
            ig, fg, gg, og = _gates(g)
            c = fg * c + ig * gg
            h = og * jnp.tanh(c)
        return h, c

    hf, _ = jax.lax.fori_loop(0, L // 8, chunk, (h0, h0))
    gb = wihb_ref[...] * xT_ref[L - 1:L, :] + b1b_ref[...]
    ib, fb, gbg, ob = _gates(gb)
    hb = ob * jnp.tanh(ib * gbg)
    a_ref[...] = jnp.concatenate([hf, hb], axis=0)


def _tree_kernel(a_ref, wih2f_ref, whh2f_ref, b2f_ref, wih2b_ref, b2b_ref,
                 wcat_ref, wlin_ref, blin_ref, out_ref, p3_ref, c3_ref):
    wih2f = wih2f_ref[...]
    whh2f = whh2f_ref[...]
    b2f = b2f_ref[...]
    b2b = b2b_ref[...]

    CH = 512
    hl_f = None
    for kc in range(N // CH):
        an = a_ref[:, kc * CH:(kc + 1) * CH]
        g1 = jnp.dot(wih2f, an, preferred_element_type=jnp.float32) + b2f
        i1, f1, g1g, o1 = _gates(g1)
        c1 = i1 * g1g
        h1 = o1 * jnp.tanh(c1)
        p = jnp.dot(whh2f, h1, preferred_element_type=jnp.float32) + b2f
        for j in range(CH // B):
            m = kc * (CH // B) + j
            p3_ref[m] = p[:, j * B:(j + 1) * B]
            c3_ref[m] = c1[:, j * B:(j + 1) * B]
        if kc == N // CH - 1:
            hl_f = h1[:, CH - B:CH]

    a_leaf = a_ref[:, N - B:N]
    gbl = jnp.dot(wih2b_ref[...], a_leaf,
                  preferred_element_type=jnp.float32) + b2b
    ibl, fbl, gblg, obl = _gates(gbl)
    hl_b = obl * jnp.tanh(ibl * gblg)
    f_leaf = jnp.concatenate([hl_f, hl_b], axis=0)

    wcat = wcat_ref[...]

    def step(j, f):
        m = M - 2 - j
        gboth = jnp.dot(wcat, f, preferred_element_type=jnp.float32)
        g2 = gboth[0:G4] + p3_ref[m]
        i2, f2, g2g, o2 = _gates(g2)
        c2 = f2 * c3_ref[m] + i2 * g2g
        h2 = o2 * jnp.tanh(c2)
        gb = gboth[G4:2 * G4] + b2b
        ib, fb, gbg, ob = _gates(gb)
        hb = ob * jnp.tanh(ib * gbg)
        return jnp.concatenate([h2, hb], axis=0)

    f0 = jax.lax.fori_loop(0, M - 1, step, f_leaf)

    lg = jnp.dot(wlin_ref[...], f0,
                 preferred_element_type=jnp.float32) + blin_ref[...]
    l0, l1 = lg[0:1], lg[1:2]
    mx = jnp.maximum(l0, l1)
    e0 = jnp.exp(l0 - mx)
    e1 = jnp.exp(l1 - mx)
    s = e0 + e1
    out_ref[...] = jnp.concatenate(
        [e0 / s, e1 / s, jnp.zeros((6, B), jnp.float32)], axis=0)


def kernel(x, w_ih1f, w_hh1f, b_ih1f, b_hh1f, w_ih1b, w_hh1b, b_ih1b, b_hh1b,
           w_ih2f, w_hh2f, b_ih2f, b_hh2f, w_ih2b, w_hh2b, b_ih2b, b_hh2b,
           w_lin, b_lin):
    f32 = jnp.float32
    xT = x[..., 0].transpose(2, 1, 0).reshape(L, N)
    b1f = (b_ih1f + b_hh1f).reshape(G4, 1)
    b1b = (b_ih1b + b_hh1b).reshape(G4, 1)
    b2f = (b_ih2f + b_hh2f).reshape(G4, 1)
    b2b = (b_ih2b + b_hh2b).reshape(G4, 1)
    wcat = jnp.concatenate([w_ih2f, w_ih2b], axis=0)
    wlin = jnp.zeros((8, 2 * H), f32).at[0:2].set(w_lin)
    blin = jnp.zeros((8, 1), f32).at[0:2, 0].set(b_lin)

    bn = N // 2
    wspec = lambda shp: pl.BlockSpec(shp, lambda i: (0, 0))
    a = pl.pallas_call(
        _lstm1_kernel,
        grid=(2,),
        in_specs=[
            pl.BlockSpec((L, bn), lambda i: (0, i)),
            wspec((G4, H)),
            wspec((G4, 1)),
            wspec((G4, 1)),
            wspec((G4, 1)),
            wspec((G4, 1)),
        ],
        out_specs=pl.BlockSpec((2 * H, bn), lambda i: (0, i)),
        out_shape=jax.ShapeDtypeStruct((2 * H, N), f32),
        compiler_params=pltpu.CompilerParams(
            dimension_semantics=("parallel",)),
    )(xT, w_hh1f, w_ih1f, b1f, w_ih1b, b1b)

    out8 = pl.pallas_call(
        _tree_kernel,
        out_shape=jax.ShapeDtypeStruct((8, B), f32),
        scratch_shapes=[
            pltpu.VMEM((M, G4, B), f32),
            pltpu.VMEM((M, H, B), f32),
        ],
    )(a, w_ih2f, w_hh2f, b2f, w_ih2b, b2b, wcat, wlin, blin)

    return out8[0:2].T


# speedup vs baseline: 2.8004x; 1.0578x over previous
import jax
import jax.numpy as jnp
from jax.experimental import pallas as pl
from jax.experimental.pallas import tpu as pltpu

H = 32
B = 64
M = 64
L = 256
N = M * B
G4 = 4 * H


def _sig(v):
    return 0.5 * jnp.tanh(0.5 * v) + 0.5


def _gates(g):
    return (_sig(g[0:H]), _sig(g[H:2 * H]), jnp.tanh(g[2 * H:3 * H]),
            _sig(g[3 * H:4 * H]))


def _lstm1_kernel(xT_ref, whh_ref, wih_ref, b1_ref, wihb_ref, b1b_ref, a_ref):
    bn = xT_ref.shape[1]
    whh = whh_ref[...]
    wih = wih_ref[...]
    b1 = b1_ref[...]
    h0 = jnp.zeros((H, bn), jnp.float32)

    def chunk(k, carry):
        h, c = carry
        xc = xT_ref[pl.ds(pl.multiple_of(k * 8, 8), 8), :]
        for s in range(8):
            xt = xc[s:s + 1, :]
            g = jnp.dot(whh, h, preferred_element_type=jnp.float32)
            g = g + (wih * xt + b1)
            ig, fg, gg, og = _gates(g)
            c = fg * c + ig * gg
            h = og * jnp.tanh(c)
        return h, c

    hf, _ = jax.lax.fori_loop(0, L // 8, chunk, (h0, h0))
    gb = wihb_ref[...] * xT_ref[L - 1:L, :] + b1b_ref[...]
    ib, fb, gbg, ob = _gates(gb)
    hb = ob * jnp.tanh(ib * gbg)
    a_ref[...] = jnp.concatenate([hf, hb], axis=0)


def _tree_kernel(a_ref, wih2f_ref, whh2f_ref, b2f_ref, wih2b_ref, b2b_ref,
                 wcat_ref, wlin_ref, blin_ref, out_ref, p3_ref, c3_ref):
    wih2f = wih2f_ref[...]
    whh2f = whh2f_ref[...]
    b2f = b2f_ref[...]
    b2b = b2b_ref[...]

    CH = 512
    hl_f = None
    for kc in range(N // CH):
        an = a_ref[:, kc * CH:(kc + 1) * CH]
        g1 = jnp.dot(wih2f, an, preferred_element_type=jnp.float32) + b2f
        i1, f1, g1g, o1 = _gates(g1)
        c1 = i1 * g1g
        h1 = o1 * jnp.tanh(c1)
        p = jnp.dot(whh2f, h1, preferred_element_type=jnp.float32) + b2f
        for j in range(CH // B):
            m = kc * (CH // B) + j
            p3_ref[m] = p[:, j * B:(j + 1) * B]
            c3_ref[m] = c1[:, j * B:(j + 1) * B]
        if kc == N // CH - 1:
            hl_f = h1[:, CH - B:CH]

    a_leaf = a_ref[:, N - B:N]
    gbl = jnp.dot(wih2b_ref[...], a_leaf,
                  preferred_element_type=jnp.float32) + b2b
    ibl, fbl, gblg, obl = _gates(gbl)
    hl_b = obl * jnp.tanh(ibl * gblg)
    f_leaf = jnp.concatenate([hl_f, hl_b], axis=0)

    wcat = wcat_ref[...]

    def step(j, f):
        m = M - 2 - j
        gboth = jnp.dot(wcat, f, preferred_element_type=jnp.float32)
        g2 = gboth[0:G4] + p3_ref[m]
        i2, f2, g2g, o2 = _gates(g2)
        c2 = f2 * c3_ref[m] + i2 * g2g
        h2 = o2 * jnp.tanh(c2)
        gb = gboth[G4:2 * G4] + b2b
        ib, fb, gbg, ob = _gates(gb)
        hb = ob * jnp.tanh(ib * gbg)
        return jnp.concatenate([h2, hb], axis=0)

    f0 = jax.lax.fori_loop(0, M - 1, step, f_leaf)

    lg = jnp.dot(wlin_ref[...], f0,
                 preferred_element_type=jnp.float32) + blin_ref[...]
    l0, l1 = lg[0:1], lg[1:2]
    mx = jnp.maximum(l0, l1)
    e0 = jnp.exp(l0 - mx)
    e1 = jnp.exp(l1 - mx)
    s = e0 + e1
    out_ref[...] = jnp.concatenate(
        [e0 / s, e1 / s, jnp.zeros((6, B), jnp.float32)], axis=0)


def kernel(x, w_ih1f, w_hh1f, b_ih1f, b_hh1f, w_ih1b, w_hh1b, b_ih1b, b_hh1b,
           w_ih2f, w_hh2f, b_ih2f, b_hh2f, w_ih2b, w_hh2b, b_ih2b, b_hh2b,
           w_lin, b_lin):
    f32 = jnp.float32
    xT = x[..., 0].transpose(2, 1, 0).reshape(L, N)
    b1f = (b_ih1f + b_hh1f).reshape(G4, 1)
    b1b = (b_ih1b + b_hh1b).reshape(G4, 1)
    b2f = (b_ih2f + b_hh2f).reshape(G4, 1)
    b2b = (b_ih2b + b_hh2b).reshape(G4, 1)
    wcat = jnp.concatenate([w_ih2f, w_ih2b], axis=0)
    wlin = jnp.zeros((8, 2 * H), f32).at[0:2].set(w_lin)
    blin = jnp.zeros((8, 1), f32).at[0:2, 0].set(b_lin)

    bn = N // 2
    wspec = lambda shp: pl.BlockSpec(shp, lambda i: (0, 0))
    a = pl.pallas_call(
        _lstm1_kernel,
        grid=(2,),
        in_specs=[
            pl.BlockSpec((L, bn), lambda i: (0, i)),
            wspec((G4, H)),
            wspec((G4, 1)),
            wspec((G4, 1)),
            wspec((G4, 1)),
            wspec((G4, 1)),
        ],
        out_specs=pl.BlockSpec((2 * H, bn), lambda i: (0, i)),
        out_shape=jax.ShapeDtypeStruct((2 * H, N), f32),
        compiler_params=pltpu.CompilerParams(
            dimension_semantics=("parallel",)),
    )(xT, w_hh1f, w_ih1f, b1f, w_ih1b, b1b)

    out8 = pl.pallas_call(
        _tree_kernel,
        out_shape=jax.ShapeDtypeStruct((8, B), f32),
        scratch_shapes=[
            pltpu.VMEM((M, G4, B), f32),
            pltpu.VMEM((M, H, B), f32),
        ],
    )(a, w_ih2f, w_hh2f, b2f, w_ih2b, b2b, wcat, wlin, blin)

    return out8[0:2].T


# speedup vs baseline: 3.3576x; 1.1990x over previous
import jax
import jax.numpy as jnp
from jax.experimental import pallas as pl
from jax.experimental.pallas import tpu as pltpu

H = 32
B = 64
M = 64
L = 256
N = M * B
G4 = 4 * H


def _sig(v):
    return 0.5 * jnp.tanh(0.5 * v) + 0.5


def _gates(g):
    return (_sig(g[0:H]), _sig(g[H:2 * H]), jnp.tanh(g[2 * H:3 * H]),
            _sig(g[3 * H:4 * H]))


def _lstm1_kernel(xT_ref, waug_ref, wihb_ref, b1b_ref, a_ref):
    bn = xT_ref.shape[1]
    h0 = jnp.zeros((H, bn), jnp.float32)
    ones8 = jnp.ones((8, bn), jnp.bfloat16)

    def chunk(k, carry):
        h2, c = carry
        xc = xT_ref[pl.ds(pl.multiple_of(k * 8, 8), 8), :]
        xc16 = xc.astype(jnp.bfloat16)
        for s in range(8):
            hx = jnp.concatenate(
                [h2.astype(jnp.bfloat16), xc16, ones8], axis=0)
            g = jnp.dot(waug_ref[s], hx,
                        preferred_element_type=jnp.float32)
            t = jnp.tanh(g)
            ti, tf = t[0:H], t[H:2 * H]
            tg, to = t[2 * H:3 * H], t[3 * H:4 * H]
            c = 0.5 * (c * (tf + 1.0) + tg * (ti + 1.0))
            h2 = (to + 1.0) * jnp.tanh(c)
        return h2, c

    h2f, _ = jax.lax.fori_loop(0, L // 8, chunk, (h0, h0))
    hf = 0.5 * h2f
    gb = wihb_ref[...] * xT_ref[L - 1:L, :] + b1b_ref[...]
    tb = jnp.tanh(gb)
    cb = 0.5 * (tb[2 * H:3 * H] * (tb[0:H] + 1.0))
    hb = 0.5 * (tb[3 * H:4 * H] + 1.0) * jnp.tanh(cb)
    a_ref[...] = jnp.concatenate([hf, hb], axis=0)


def _tree_kernel(a_ref, wih2f_ref, whh2f_ref, b2f_ref, wih2b_ref, b2b_ref,
                 wcat_ref, wlin_ref, blin_ref, out_ref, p3_ref, c3_ref):
    wih2f = wih2f_ref[...]
    whh2f = whh2f_ref[...]
    b2f = b2f_ref[...]
    b2b = b2b_ref[...]

    CH = 512
    hl_f = None
    for kc in range(N // CH):
        an = a_ref[:, kc * CH:(kc + 1) * CH]
        g1 = jnp.dot(wih2f, an, preferred_element_type=jnp.float32) + b2f
        i1, f1, g1g, o1 = _gates(g1)
        c1 = i1 * g1g
        h1 = o1 * jnp.tanh(c1)
        p = jnp.dot(whh2f, h1, preferred_element_type=jnp.float32) + b2f
        for j in range(CH // B):
            m = kc * (CH // B) + j
            p3_ref[m] = p[:, j * B:(j + 1) * B]
            c3_ref[m] = c1[:, j * B:(j + 1) * B]
        if kc == N // CH - 1:
            hl_f = h1[:, CH - B:CH]

    a_leaf = a_ref[:, N - B:N]
    gbl = jnp.dot(wih2b_ref[...], a_leaf,
                  preferred_element_type=jnp.float32) + b2b
    ibl, fbl, gblg, obl = _gates(gbl)
    hl_b = obl * jnp.tanh(ibl * gblg)
    f_leaf = jnp.concatenate([hl_f, hl_b], axis=0)

    wcat = wcat_ref[...]

    def step(j, f):
        m = M - 2 - j
        gboth = jnp.dot(wcat, f, preferred_element_type=jnp.float32)
        g2 = gboth[0:G4] + p3_ref[m]
        i2, f2, g2g, o2 = _gates(g2)
        c2 = f2 * c3_ref[m] + i2 * g2g
        h2 = o2 * jnp.tanh(c2)
        gb = gboth[G4:2 * G4] + b2b
        ib, fb, gbg, ob = _gates(gb)
        hb = ob * jnp.tanh(ib * gbg)
        return jnp.concatenate([h2, hb], axis=0)

    f0 = jax.lax.fori_loop(0, M - 1, step, f_leaf)

    lg = jnp.dot(wlin_ref[...], f0,
                 preferred_element_type=jnp.float32) + blin_ref[...]
    l0, l1 = lg[0:1], lg[1:2]
    mx = jnp.maximum(l0, l1)
    e0 = jnp.exp(l0 - mx)
    e1 = jnp.exp(l1 - mx)
    s = e0 + e1
    out_ref[...] = jnp.concatenate(
        [e0 / s, e1 / s, jnp.zeros((6, B), jnp.float32)], axis=0)


def kernel(x, w_ih1f, w_hh1f, b_ih1f, b_hh1f, w_ih1b, w_hh1b, b_ih1b, b_hh1b,
           w_ih2f, w_hh2f, b_ih2f, b_hh2f, w_ih2b, w_hh2b, b_ih2b, b_hh2b,
           w_lin, b_lin):
    f32 = jnp.float32
    xT = x[..., 0].transpose(2, 1, 0).reshape(L, N)
    b1f = (b_ih1f + b_hh1f).reshape(G4, 1)
    b1b = (b_ih1b + b_hh1b).reshape(G4, 1)
    rsc = jnp.concatenate([jnp.full((2 * H, 1), 0.5, f32),
                           jnp.ones((H, 1), f32),
                           jnp.full((H, 1), 0.5, f32)], axis=0)
    whh_e = rsc * w_hh1f * 0.5
    wih_e = rsc * w_ih1f
    b1_e = rsc * b1f
    b_hi = b1_e.astype(jnp.bfloat16).astype(f32)
    b_lo = b1_e - b_hi
    waug = jnp.zeros((8, G4, 48), f32)
    waug = waug.at[:, :, 0:H].set(whh_e[None])
    waug = waug.at[:, :, 40].set(b_hi[:, 0][None])
    waug = waug.at[:, :, 41].set(b_lo[:, 0][None])
    for s in range(8):
        waug = waug.at[s, :, H + s].set(wih_e[:, 0])
    waug = waug.astype(jnp.bfloat16)
    wihb_e = rsc * w_ih1b
    b1b_e = rsc * b1b
    b2f = (b_ih2f + b_hh2f).reshape(G4, 1)
    b2b = (b_ih2b + b_hh2b).reshape(G4, 1)
    wcat = jnp.concatenate([w_ih2f, w_ih2b], axis=0)
    wlin = jnp.zeros((8, 2 * H), f32).at[0:2].set(w_lin)
    blin = jnp.zeros((8, 1), f32).at[0:2, 0].set(b_lin)

    bn = N // 2
    wspec = lambda shp: pl.BlockSpec(shp, lambda i: (0, 0))
    a = pl.pallas_call(
        _lstm1_kernel,
        grid=(2,),
        in_specs=[
            pl.BlockSpec((L, bn), lambda i: (0, i)),
            pl.BlockSpec((8, G4, 48), lambda i: (0, 0, 0)),
            wspec((G4, 1)),
            wspec((G4, 1)),
        ],
        out_specs=pl.BlockSpec((2 * H, bn), lambda i: (0, i)),
        out_shape=jax.ShapeDtypeStruct((2 * H, N), f32),
        compiler_params=pltpu.CompilerParams(
            dimension_semantics=("parallel",)),
    )(xT, waug, wihb_e, b1b_e)

    out8 = pl.pallas_call(
        _tree_kernel,
        out_shape=jax.ShapeDtypeStruct((8, B), f32),
        scratch_shapes=[
            pltpu.VMEM((M, G4, B), f32),
            pltpu.VMEM((M, H, B), f32),
        ],
    )(a, w_ih2f, w_hh2f, b2f, w_ih2b, b2b, wcat, wlin, blin)

    return out8[0:2].T


# speedup vs baseline: 3.8739x; 1.1538x over previous
import jax
import jax.numpy as jnp
from jax.experimental import pallas as pl
from jax.experimental.pallas import tpu as pltpu

H = 32
B = 64
M = 64
L = 256
N = M * B
G4 = 4 * H


def _lstm1_kernel(xT_ref, waug_ref, wihb_ref, b1b_ref, a_ref):
    bn = xT_ref.shape[1]
    h0 = jnp.zeros((H, bn), jnp.float32)
    ones8 = jnp.ones((8, bn), jnp.bfloat16)

    def chunk(k, carry):
        h2, c = carry
        xc = xT_ref[pl.ds(pl.multiple_of(k * 8, 8), 8), :]
        xc16 = xc.astype(jnp.bfloat16)
        for s in range(8):
            hx = jnp.concatenate(
                [h2.astype(jnp.bfloat16), xc16, ones8], axis=0)
            g = jnp.dot(waug_ref[s], hx,
                        preferred_element_type=jnp.float32)
            t = jnp.tanh(g)
            ti, tf = t[0:H], t[H:2 * H]
            tg, to = t[2 * H:3 * H], t[3 * H:4 * H]
            c = 0.5 * (c * (tf + 1.0) + tg * (ti + 1.0))
            h2 = (to + 1.0) * jnp.tanh(c)
        return h2, c

    h2f, _ = jax.lax.fori_loop(0, L // 8, chunk, (h0, h0))
    hf = 0.5 * h2f
    gb = wihb_ref[...] * xT_ref[L - 1:L, :] + b1b_ref[...]
    tb = jnp.tanh(gb)
    cb = 0.5 * (tb[2 * H:3 * H] * (tb[0:H] + 1.0))
    hb = 0.5 * (tb[3 * H:4 * H] + 1.0) * jnp.tanh(cb)
    a_ref[...] = jnp.concatenate([hf, hb], axis=0).astype(jnp.bfloat16)


def _tree_kernel(a_ref, wih2f_ref, whh2f_ref, wih2b_ref, wcat_ref,
                 wlin_ref, blin_ref, out_ref, p3_ref, c3_ref):
    bf = jnp.bfloat16

    CH = 512
    ones16c = jnp.ones((16, CH), bf)
    hl_f = None
    for kc in range(N // CH):
        an = a_ref[:, kc * CH:(kc + 1) * CH]
        g1 = jnp.dot(wih2f_ref[...], jnp.concatenate([an, ones16c], axis=0),
                     preferred_element_type=jnp.float32)
        t1 = jnp.tanh(g1)
        c1 = 0.5 * (t1[2 * H:3 * H] * (t1[0:H] + 1.0))
        h1x = (t1[3 * H:4 * H] + 1.0) * jnp.tanh(c1)
        p = jnp.dot(whh2f_ref[...],
                    jnp.concatenate([h1x.astype(bf), ones16c], axis=0),
                    preferred_element_type=jnp.float32)
        for j in range(CH // B):
            m = kc * (CH // B) + j
            p3_ref[m] = p[:, j * B:(j + 1) * B]
            c3_ref[m] = c1[:, j * B:(j + 1) * B]
        if kc == N // CH - 1:
            hl_f = h1x[:, CH - B:CH]

    ones16 = jnp.ones((16, B), bf)
    a_leaf = a_ref[:, N - B:N]
    gbl = jnp.dot(wih2b_ref[...],
                  jnp.concatenate([a_leaf, ones16], axis=0),
                  preferred_element_type=jnp.float32)
    tbl = jnp.tanh(gbl)
    cbl = 0.5 * (tbl[2 * H:3 * H] * (tbl[0:H] + 1.0))
    hl_b = (tbl[3 * H:4 * H] + 1.0) * jnp.tanh(cbl)
    f2x = jnp.concatenate([hl_f, hl_b], axis=0)

    def step(j, f2):
        m = M - 2 - j
        gboth = jnp.dot(wcat_ref[...],
                        jnp.concatenate([f2.astype(bf), ones16], axis=0),
                        preferred_element_type=jnp.float32)
        t = jnp.tanh(gboth[0:G4] + p3_ref[m])
        c2 = 0.5 * (c3_ref[m] * (t[H:2 * H] + 1.0)
                    + t[2 * H:3 * H] * (t[0:H] + 1.0))
        h2x = (t[3 * H:4 * H] + 1.0) * jnp.tanh(c2)
        tb = jnp.tanh(gboth[G4:2 * G4])
        cb = 0.5 * (tb[2 * H:3 * H] * (tb[0:H] + 1.0))
        hb2x = (tb[3 * H:4 * H] + 1.0) * jnp.tanh(cb)
        return jnp.concatenate([h2x, hb2x], axis=0)

    f0x = jax.lax.fori_loop(0, M - 1, step, f2x)

    lg = jnp.dot(wlin_ref[...], f0x,
                 preferred_element_type=jnp.float32) + blin_ref[...]
    l0, l1 = lg[0:1], lg[1:2]
    mx = jnp.maximum(l0, l1)
    e0 = jnp.exp(l0 - mx)
    e1 = jnp.exp(l1 - mx)
    s = e0 + e1
    out_ref[...] = jnp.concatenate(
        [e0 / s, e1 / s, jnp.zeros((6, B), jnp.float32)], axis=0)


def kernel(x, w_ih1f, w_hh1f, b_ih1f, b_hh1f, w_ih1b, w_hh1b, b_ih1b, b_hh1b,
           w_ih2f, w_hh2f, b_ih2f, b_hh2f, w_ih2b, w_hh2b, b_ih2b, b_hh2b,
           w_lin, b_lin):
    f32 = jnp.float32
    xT = x[..., 0].transpose(2, 1, 0).reshape(L, N)
    b1f = (b_ih1f + b_hh1f).reshape(G4, 1)
    b1b = (b_ih1b + b_hh1b).reshape(G4, 1)
    rsc = jnp.concatenate([jnp.full((2 * H, 1), 0.5, f32),
                           jnp.ones((H, 1), f32),
                           jnp.full((H, 1), 0.5, f32)], axis=0)
    whh_e = rsc * w_hh1f * 0.5
    wih_e = rsc * w_ih1f
    b1_e = rsc * b1f
    b_hi = b1_e.astype(jnp.bfloat16).astype(f32)
    b_lo = b1_e - b_hi
    waug = jnp.zeros((8, G4, 48), f32)
    waug = waug.at[:, :, 0:H].set(whh_e[None])
    waug = waug.at[:, :, 40].set(b_hi[:, 0][None])
    waug = waug.at[:, :, 41].set(b_lo[:, 0][None])
    for s in range(8):
        waug = waug.at[s, :, H + s].set(wih_e[:, 0])
    waug = waug.astype(jnp.bfloat16)
    wihb_e = rsc * w_ih1b
    b1b_e = rsc * b1b
    b2f = (b_ih2f + b_hh2f).reshape(G4, 1)
    b2b = (b_ih2b + b_hh2b).reshape(G4, 1)

    def _aug(w_e, b_e, k):
        aug = jnp.zeros((w_e.shape[0], k), f32)
        b_hi = b_e.astype(jnp.bfloat16).astype(f32)
        aug = aug.at[:, 0:w_e.shape[1]].set(w_e)
        aug = aug.at[:, w_e.shape[1]].set(b_hi[:, 0])
        aug = aug.at[:, w_e.shape[1] + 1].set((b_e - b_hi)[:, 0])
        return aug.astype(jnp.bfloat16)

    wih2f_a = _aug(rsc * w_ih2f, rsc * b2f, 80)
    whh2f_a = _aug(rsc * w_hh2f * 0.5, rsc * b2f, 48)
    wih2b_a = _aug(rsc * w_ih2b, rsc * b2b, 80)
    wcat_e = jnp.concatenate([rsc * w_ih2f, rsc * w_ih2b], axis=0) * 0.5
    bcat = jnp.concatenate([jnp.zeros((G4, 1), f32), rsc * b2b], axis=0)
    wcat_a = _aug(wcat_e, bcat, 80)
    wlin = (jnp.zeros((8, 2 * H), f32).at[0:2].set(w_lin)) * 0.5
    blin = jnp.zeros((8, 1), f32).at[0:2, 0].set(b_lin)

    bn = N
    wspec = lambda shp: pl.BlockSpec(shp, lambda i: (0, 0))
    a = pl.pallas_call(
        _lstm1_kernel,
        grid=(1,),
        in_specs=[
            pl.BlockSpec((L, bn), lambda i: (0, i)),
            pl.BlockSpec((8, G4, 48), lambda i: (0, 0, 0)),
            wspec((G4, 1)),
            wspec((G4, 1)),
        ],
        out_specs=pl.BlockSpec((2 * H, bn), lambda i: (0, i)),
        out_shape=jax.ShapeDtypeStruct((2 * H, N), jnp.bfloat16),
        compiler_params=pltpu.CompilerParams(
            dimension_semantics=("parallel",)),
    )(xT, waug, wihb_e, b1b_e)

    out8 = pl.pallas_call(
        _tree_kernel,
        out_shape=jax.ShapeDtypeStruct((8, B), f32),
        scratch_shapes=[
            pltpu.VMEM((M, G4, B), f32),
            pltpu.VMEM((M, H, B), f32),
        ],
    )(a, wih2f_a, whh2f_a, wih2b_a, wcat_a, wlin, blin)

    return out8[0:2].T


# speedup vs baseline: 4.0676x; 1.0500x over previous
import jax
import jax.numpy as jnp
from jax.experimental import pallas as pl
from jax.experimental.pallas import tpu as pltpu

H = 32
B = 64
M = 64
L = 256
N = M * B
G4 = 4 * H


def _fused_kernel(xT_ref, waug_ref, wihb_ref, b1b_ref,
                  wih2f_ref, whh2f_ref, wih2b_ref, wcat_ref,
                  wlin_ref, blin_ref, out_ref, p3_ref, c3_ref):
    bf = jnp.bfloat16
    h0 = jnp.zeros((H, N), jnp.float32)
    ones8 = jnp.ones((8, N), bf)

    def chunk(k, carry):
        h2, c = carry
        xc = xT_ref[pl.ds(pl.multiple_of(k * 8, 8), 8), :]
        xc16 = xc.astype(bf)
        for s in range(8):
            hx = jnp.concatenate([h2.astype(bf), xc16, ones8], axis=0)
            g = jnp.dot(waug_ref[s], hx,
                        preferred_element_type=jnp.float32)
            t = jnp.tanh(g)
            ti, tf = t[0:H], t[H:2 * H]
            tg, to = t[2 * H:3 * H], t[3 * H:4 * H]
            c = 0.5 * (c * (tf + 1.0) + tg * (ti + 1.0))
            h2 = (to + 1.0) * jnp.tanh(c)
        return h2, c

    h2f, _ = jax.lax.fori_loop(0, L // 8, chunk, (h0, h0))
    hf = 0.5 * h2f
    gb = wihb_ref[...] * xT_ref[L - 1:L, :] + b1b_ref[...]
    tb1 = jnp.tanh(gb)
    cb1 = 0.5 * (tb1[2 * H:3 * H] * (tb1[0:H] + 1.0))
    hb1 = 0.5 * (tb1[3 * H:4 * H] + 1.0) * jnp.tanh(cb1)
    a = jnp.concatenate([hf, hb1], axis=0).astype(bf)

    CH = 512
    ones16c = jnp.ones((16, CH), bf)
    hl_f = None
    for kc in range(N // CH):
        an = a[:, kc * CH:(kc + 1) * CH]
        g1 = jnp.dot(wih2f_ref[...], jnp.concatenate([an, ones16c], axis=0),
                     preferred_element_type=jnp.float32)
        t1 = jnp.tanh(g1)
        c1 = 0.5 * (t1[2 * H:3 * H] * (t1[0:H] + 1.0))
        h1x = (t1[3 * H:4 * H] + 1.0) * jnp.tanh(c1)
        p = jnp.dot(whh2f_ref[...],
                    jnp.concatenate([h1x.astype(bf), ones16c], axis=0),
                    preferred_element_type=jnp.float32)
        for j in range(CH // B):
            m = kc * (CH // B) + j
            p3_ref[m] = p[:, j * B:(j + 1) * B]
            c3_ref[m] = c1[:, j * B:(j + 1) * B]
        if kc == N // CH - 1:
            hl_f = h1x[:, CH - B:CH]

    ones16 = jnp.ones((16, B), bf)
    a_leaf = a[:, N - B:N]
    gbl = jnp.dot(wih2b_ref[...],
                  jnp.concatenate([a_leaf, ones16], axis=0),
                  preferred_element_type=jnp.float32)
    tbl = jnp.tanh(gbl)
    cbl = 0.5 * (tbl[2 * H:3 * H] * (tbl[0:H] + 1.0))
    hl_b = (tbl[3 * H:4 * H] + 1.0) * jnp.tanh(cbl)
    f2x = jnp.concatenate([hl_f, hl_b], axis=0)

    def step(j, f2):
        m = M - 2 - j
        gboth = jnp.dot(wcat_ref[...],
                        jnp.concatenate([f2.astype(bf), ones16], axis=0),
                        preferred_element_type=jnp.float32)
        t = jnp.tanh(gboth[0:G4] + p3_ref[m])
        c2 = 0.5 * (c3_ref[m] * (t[H:2 * H] + 1.0)
                    + t[2 * H:3 * H] * (t[0:H] + 1.0))
        h2x = (t[3 * H:4 * H] + 1.0) * jnp.tanh(c2)
        tb = jnp.tanh(gboth[G4:2 * G4])
        cb = 0.5 * (tb[2 * H:3 * H] * (tb[0:H] + 1.0))
        hb2x = (tb[3 * H:4 * H] + 1.0) * jnp.tanh(cb)
        return jnp.concatenate([h2x, hb2x], axis=0)

    f0x = jax.lax.fori_loop(0, M - 1, step, f2x)

    lg = jnp.dot(wlin_ref[...], f0x,
                 preferred_element_type=jnp.float32) + blin_ref[...]
    l0, l1 = lg[0:1], lg[1:2]
    mx = jnp.maximum(l0, l1)
    e0 = jnp.exp(l0 - mx)
    e1 = jnp.exp(l1 - mx)
    s = e0 + e1
    out_ref[...] = jnp.concatenate(
        [e0 / s, e1 / s, jnp.zeros((6, B), jnp.float32)], axis=0)


def kernel(x, w_ih1f, w_hh1f, b_ih1f, b_hh1f, w_ih1b, w_hh1b, b_ih1b, b_hh1b,
           w_ih2f, w_hh2f, b_ih2f, b_hh2f, w_ih2b, w_hh2b, b_ih2b, b_hh2b,
           w_lin, b_lin):
    f32 = jnp.float32
    xT = x[..., 0].transpose(2, 1, 0).reshape(L, N)
    b1f = (b_ih1f + b_hh1f).reshape(G4, 1)
    b1b = (b_ih1b + b_hh1b).reshape(G4, 1)
    rsc = jnp.concatenate([jnp.full((2 * H, 1), 0.5, f32),
                           jnp.ones((H, 1), f32),
                           jnp.full((H, 1), 0.5, f32)], axis=0)
    whh_e = rsc * w_hh1f * 0.5
    wih_e = rsc * w_ih1f
    b1_e = rsc * b1f
    b_hi = b1_e.astype(jnp.bfloat16).astype(f32)
    b_lo = b1_e - b_hi
    xsel = (jnp.arange(8)[:, None, None] + H == jnp.arange(48)[None, None, :])
    waug = jnp.where(xsel, wih_e[None], 0.0)
    bias2 = jnp.concatenate(
        [b_hi, b_lo, jnp.zeros((G4, 6), f32)], axis=1)
    body = jnp.concatenate([whh_e, jnp.zeros((G4, 8), f32), bias2], axis=1)
    waug = (waug + body[None]).astype(jnp.bfloat16)
    wihb_e = rsc * w_ih1b
    b1b_e = rsc * b1b
    b2f = (b_ih2f + b_hh2f).reshape(G4, 1)
    b2b = (b_ih2b + b_hh2b).reshape(G4, 1)

    def _aug(w_e, b_e, k):
        bh = b_e.astype(jnp.bfloat16).astype(f32)
        aug = jnp.concatenate(
            [w_e, bh, b_e - bh,
             jnp.zeros((w_e.shape[0], k - w_e.shape[1] - 2), f32)], axis=1)
        return aug.astype(jnp.bfloat16)

    wih2f_a = _aug(rsc * w_ih2f, rsc * b2f, 80)
    whh2f_a = _aug(rsc * w_hh2f * 0.5, rsc * b2f, 48)
    wih2b_a = _aug(rsc * w_ih2b, rsc * b2b, 80)
    wcat_e = jnp.concatenate([rsc * w_ih2f, rsc * w_ih2b], axis=0) * 0.5
    bcat = jnp.concatenate([jnp.zeros((G4, 1), f32), rsc * b2b], axis=0)
    wcat_a = _aug(wcat_e, bcat, 80)
    wlin = (jnp.zeros((8, 2 * H), f32).at[0:2].set(w_lin)) * 0.5
    blin = jnp.zeros((8, 1), f32).at[0:2, 0].set(b_lin)

    out8 = pl.pallas_call(
        _fused_kernel,
        out_shape=jax.ShapeDtypeStruct((8, B), f32),
        scratch_shapes=[
            pltpu.VMEM((M, G4, B), f32),
            pltpu.VMEM((M, H, B), f32),
        ],
    )(xT, waug, wihb_e, b1b_e, wih2f_a, whh2f_a, wih2b_a, wcat_a, wlin, blin)

    return out8[0:2].T


# speedup vs baseline: 4.2913x; 1.0550x over previous
import jax
import jax.numpy as jnp
from jax.experimental import pallas as pl
from jax.experimental.pallas import tpu as pltpu

H = 32
B = 64
M = 64
L = 256
N = M * B
G4 = 4 * H


def _row_prescale():
    r = jax.lax.broadcasted_iota(jnp.int32, (G4, 1), 0)
    return jnp.where((r < 2 * H) | (r >= 3 * H), 0.5, 1.0)


def _split_bias(b_e):
    bh = b_e.astype(jnp.bfloat16).astype(jnp.float32)
    return bh, b_e - bh


def _cell_tail(t, c_prev):
    c2 = 0.5 * (c_prev * (t[H:2 * H] + 1.0)
                + t[2 * H:3 * H] * (t[0:H] + 1.0))
    return (t[3 * H:4 * H] + 1.0) * jnp.tanh(c2), c2


def _fused_kernel(xT_ref, whh1f_ref, wih1f_ref, bih1f_ref, bhh1f_ref,
                  wih1b_ref, bih1b_ref, bhh1b_ref,
                  wih2f_ref, whh2f_ref, bih2f_ref, bhh2f_ref,
                  wih2b_ref, bih2b_ref, bhh2b_ref,
                  wlin_ref, blin_ref, out_ref, waug_ref, p3_ref, c3_ref):
    bf = jnp.bfloat16
    f32 = jnp.float32
    rsc = _row_prescale()

    whh_e = rsc * whh1f_ref[...] * 0.5
    wih_e = rsc * wih1f_ref[...]
    b_hi, b_lo = _split_bias(rsc * (bih1f_ref[...] + bhh1f_ref[...]))
    z1 = jnp.zeros((G4, 1), f32)
    for s in range(8):
        xcols = [wih_e if j == s else z1 for j in range(8)]
        waug_ref[s] = jnp.concatenate(
            [whh_e] + xcols + [b_hi, b_lo] + [z1] * 6, axis=1).astype(bf)

    h0 = jnp.zeros((H, N), f32)
    ones8 = jnp.ones((8, N), bf)

    def chunk(k, carry):
        h2, c = carry
        xc = xT_ref[pl.ds(pl.multiple_of(k * 32, 32), 32), :]
        for s in range(32):
            xpart = xc[(s // 8) * 8:(s // 8) * 8 + 8]
            hx = jnp.concatenate([h2.astype(bf), xpart, ones8], axis=0)
            g = jnp.dot(waug_ref[s % 8], hx,
                        preferred_element_type=f32)
            h2, c = _cell_tail(jnp.tanh(g), c)
        return h2, c

    h2f, _ = jax.lax.fori_loop(0, L // 32, chunk, (h0, h0))
    hf = 0.5 * h2f
    gb = (rsc * wih1b_ref[...]) * xT_ref[L - 1:L, :].astype(f32) \
        + rsc * (bih1b_ref[...] + bhh1b_ref[...])
    tb1 = jnp.tanh(gb)
    cb1 = 0.5 * (tb1[2 * H:3 * H] * (tb1[0:H] + 1.0))
    hb1 = 0.5 * (tb1[3 * H:4 * H] + 1.0) * jnp.tanh(cb1)
    a = jnp.concatenate([hf, hb1], axis=0).astype(bf)

    b2f_e = rsc * (bih2f_ref[...] + bhh2f_ref[...])
    b2b_e = rsc * (bih2b_ref[...] + bhh2b_ref[...])
    zpad = jnp.zeros((G4, 14), f32)

    def _aug(w_e, b_e):
        bh, bl = _split_bias(b_e)
        return jnp.concatenate([w_e, bh, bl, zpad[:, :78 - w_e.shape[1]]],
                               axis=1).astype(bf)

    wih2f_a = _aug(rsc * wih2f_ref[...], b2f_e)
    whh2f_a = _aug(rsc * whh2f_ref[...] * 0.5, b2f_e)
    wih2b_a = _aug(rsc * wih2b_ref[...], b2b_e)
    wcat_a = jnp.concatenate(
        [_aug(rsc * wih2f_ref[...] * 0.5, jnp.zeros((G4, 1), f32)),
         _aug(rsc * wih2b_ref[...] * 0.5, b2b_e)], axis=0)

    CH = 512
    ones16c = jnp.ones((16, CH), bf)
    hl_f = None
    for kc in range(N // CH):
        an = a[:, kc * CH:(kc + 1) * CH]
        g1 = jnp.dot(wih2f_a, jnp.concatenate([an, ones16c], axis=0),
                     preferred_element_type=f32)
        t1 = jnp.tanh(g1)
        c1 = 0.5 * (t1[2 * H:3 * H] * (t1[0:H] + 1.0))
        h1x = (t1[3 * H:4 * H] + 1.0) * jnp.tanh(c1)
        p = jnp.dot(whh2f_a,
                    jnp.concatenate([h1x.astype(bf), ones16c], axis=0),
                    preferred_element_type=f32)
        for j in range(CH // B):
            m = kc * (CH // B) + j
            p3_ref[m] = p[:, j * B:(j + 1) * B]
            c3_ref[m] = c1[:, j * B:(j + 1) * B]
        if kc == N // CH - 1:
            hl_f = h1x[:, CH - B:CH]

    ones16 = jnp.ones((16, B), bf)
    gbl = jnp.dot(wih2b_a,
                  jnp.concatenate([a[:, N - B:N], ones16], axis=0),
                  preferred_element_type=f32)
    tbl = jnp.tanh(gbl)
    cbl = 0.5 * (tbl[2 * H:3 * H] * (tbl[0:H] + 1.0))
    hl_b = (tbl[3 * H:4 * H] + 1.0) * jnp.tanh(cbl)
    f2x = jnp.concatenate([hl_f, hl_b], axis=0)

    def step(j, f2):
        m = M - 2 - j
        gboth = jnp.dot(wcat_a,
                        jnp.concatenate([f2.astype(bf), ones16], axis=0),
                        preferred_element_type=f32)
        h2x, _ = _cell_tail(jnp.tanh(gboth[0:G4] + p3_ref[m]), c3_ref[m])
        tb = jnp.tanh(gboth[G4:2 * G4])
        cb = 0.5 * (tb[2 * H:3 * H] * (tb[0:H] + 1.0))
        hb2x = (tb[3 * H:4 * H] + 1.0) * jnp.tanh(cb)
        return jnp.concatenate([h2x, hb2x], axis=0)

    f0x = jax.lax.fori_loop(0, M - 1, step, f2x)

    lg = jnp.dot(0.5 * wlin_ref[...], f0x,
                 preferred_element_type=f32) + blin_ref[...]
    l0, l1 = lg[0:1], lg[1:2]
    mx = jnp.maximum(l0, l1)
    e0 = jnp.exp(l0 - mx)
    e1 = jnp.exp(l1 - mx)
    s = e0 + e1
    out_ref[...] = jnp.concatenate(
        [e0 / s, e1 / s, jnp.zeros((6, B), jnp.float32)], axis=0)


def kernel(x, w_ih1f, w_hh1f, b_ih1f, b_hh1f, w_ih1b, w_hh1b, b_ih1b, b_hh1b,
           w_ih2f, w_hh2f, b_ih2f, b_hh2f, w_ih2b, w_hh2b, b_ih2b, b_hh2b,
           w_lin, b_lin):
    f32 = jnp.float32
    xT = x[..., 0].astype(jnp.bfloat16).transpose(2, 1, 0).reshape(L, N)
    out8 = pl.pallas_call(
        _fused_kernel,
        out_shape=jax.ShapeDtypeStruct((8, B), f32),
        scratch_shapes=[
            pltpu.VMEM((8, G4, 48), jnp.bfloat16),
            pltpu.VMEM((M, G4, B), f32),
            pltpu.VMEM((M, H, B), f32),
        ],
    )(xT,
      w_hh1f, w_ih1f, b_ih1f.reshape(G4, 1), b_hh1f.reshape(G4, 1),
      w_ih1b, b_ih1b.reshape(G4, 1), b_hh1b.reshape(G4, 1),
      w_ih2f, w_hh2f, b_ih2f.reshape(G4, 1), b_hh2f.reshape(G4, 1),
      w_ih2b, b_ih2b.reshape(G4, 1), b_hh2b.reshape(G4, 1),
      w_lin, b_lin.reshape(2, 1))

    return out8[0:2].T
